# Initial kernel scaffold; baseline (speedup 1.0000x reference)
import jax
import jax.numpy as jnp
from jax.experimental import pallas as pl


def kernel(x, edge_index, W1, b1, W2, b2):
    raise NotImplementedError("write your pallas kernel here")


# speedup vs baseline: 26.1025x; 26.1025x over previous
import functools

import jax
import jax.numpy as jnp
from jax import lax
from jax.experimental import pallas as pl
from jax.experimental.pallas import tpu as pltpu, tpu_sc as plsc

N = 10000
E = 320000
D = 128
NC, NS = 2, 16
NW = NC * NS
CH = 125
ER = E // CH
NCH = ER // NW
NPAD = 10240
SROW = NPAD // NS

BLK = 1000
NB = N // BLK

_mesh = plsc.VectorSubcoreMesh(
    core_axis_name="c", subcore_axis_name="s", num_cores=NC, num_subcores=NS
)


@functools.partial(
    pl.kernel,
    out_type=jax.ShapeDtypeStruct((NC, N), jnp.float32),
    mesh=_mesh,
    scratch_types=[
        pltpu.VMEM((NCH, CH), jnp.int32),
        pltpu.VMEM((128,), jnp.float32),
        pltpu.VMEM((N,), jnp.float32),
        pltpu.VMEM_SHARED((N,), jnp.float32),
    ],
)
def _count_k(dst_hbm, out_hbm, idx_v, ones_v, zero_v, cnt_sh):
    c = lax.axis_index("c")
    s = lax.axis_index("s")
    w = c * NS + s
    for k in range(128 // 16):
        ones_v[pl.ds(k * 16, 16)] = jnp.ones((16,), jnp.float32)

    @pl.when(s == 0)
    def _zero():
        def zb(i, carry):
            zero_v[pl.ds(i * 16, 16)] = jnp.zeros((16,), jnp.float32)
            return carry

        lax.fori_loop(0, N // 16, zb, 0)
        pltpu.sync_copy(zero_v, cnt_sh)

    plsc.subcore_barrier()
    pltpu.sync_copy(dst_hbm.at[pl.ds(w * NCH, NCH)], idx_v)

    def body(j, carry):
        pltpu.sync_copy(ones_v.at[pl.ds(0, CH)], cnt_sh.at[idx_v.at[j]], add=True)
        return carry

    lax.fori_loop(0, NCH, body, 0)
    plsc.subcore_barrier()

    @pl.when(s == 0)
    def _out():
        pltpu.sync_copy(cnt_sh, out_hbm.at[c])


DH = D // 2
ECH = ER // NS


@functools.partial(
    pl.kernel,
    out_type=jax.ShapeDtypeStruct((NC, NPAD, DH), jnp.float32),
    mesh=_mesh,
    scratch_types=[
        pltpu.VMEM((ECH, CH), jnp.int32),
        pltpu.VMEM((ECH, CH), jnp.int32),
        pltpu.VMEM((CH, DH), jnp.float32),
        pltpu.VMEM((128, DH), jnp.float32),
        pltpu.VMEM_SHARED((NPAD, DH), jnp.float32),
        pltpu.SemaphoreType.DMA,
    ],
    compiler_params=pltpu.CompilerParams(use_tc_tiling_on_sc=False),
)
def _agg_k(src_hbm, dst_hbm, g0_hbm, g1_hbm, out_hbm, src_v, dst_v, rows_v, zbuf_v, acc_sh, sem):
    c = lax.axis_index("c")
    s = lax.axis_index("s")

    def zr(i, carry):
        def zk(k, carry2):
            zbuf_v[i, pl.ds(k * 16, 16)] = jnp.zeros((16,), jnp.float32)
            return carry2

        lax.fori_loop(0, DH // 16, zk, 0)
        return carry

    lax.fori_loop(0, 128, zr, 0)
    for t in range(SROW // 128):
        pltpu.sync_copy(zbuf_v, acc_sh.at[pl.ds(s * SROW + t * 128, 128)])
    plsc.subcore_barrier()

    pltpu.sync_copy(src_hbm.at[pl.ds(s * ECH, ECH)], src_v)
    pltpu.sync_copy(dst_hbm.at[pl.ds(s * ECH, ECH)], dst_v)

    def _edge_loop(g_hbm):
        def body(j, carry):
            pltpu.async_copy(g_hbm.at[src_v.at[j]], rows_v, sem).wait()
            pltpu.sync_copy(rows_v, acc_sh.at[dst_v.at[j]], add=True)
            return carry

        lax.fori_loop(0, ECH, body, 0)

    @pl.when(c == 0)
    def _half0():
        _edge_loop(g0_hbm)

    @pl.when(c == 1)
    def _half1():
        _edge_loop(g1_hbm)

    plsc.subcore_barrier()
    pltpu.sync_copy(
        acc_sh.at[pl.ds(s * SROW, SROW)], out_hbm.at[c, pl.ds(s * SROW, SROW)]
    )


@functools.partial(
    pl.kernel,
    out_type=jax.ShapeDtypeStruct((NC, N), jnp.float32),
    mesh=_mesh,
    scratch_types=[
        pltpu.VMEM((NCH, CH), jnp.int32),
        pltpu.VMEM((NCH, CH), jnp.int32),
        pltpu.VMEM((CH,), jnp.float32),
        pltpu.VMEM((N,), jnp.float32),
        pltpu.VMEM_SHARED((N,), jnp.float32),
        pltpu.SemaphoreType.DMA,
    ],
)
def _sagg_k(src_hbm, dst_hbm, q_hbm, out_hbm, src_v, dst_v, vals_v, zero_v, acc_sh, sem):
    c = lax.axis_index("c")
    s = lax.axis_index("s")
    w = c * NS + s

    @pl.when(s == 0)
    def _zero():
        def zb(i, carry):
            zero_v[pl.ds(i * 16, 16)] = jnp.zeros((16,), jnp.float32)
            return carry

        lax.fori_loop(0, N // 16, zb, 0)
        pltpu.sync_copy(zero_v, acc_sh)

    plsc.subcore_barrier()
    pltpu.sync_copy(src_hbm.at[pl.ds(w * NCH, NCH)], src_v)
    pltpu.sync_copy(dst_hbm.at[pl.ds(w * NCH, NCH)], dst_v)

    def body(j, carry):
        pltpu.async_copy(q_hbm.at[src_v.at[j]], vals_v, sem).wait()
        pltpu.sync_copy(vals_v, acc_sh.at[dst_v.at[j]], add=True)
        return carry

    lax.fori_loop(0, NCH, body, 0)
    plsc.subcore_barrier()

    @pl.when(s == 0)
    def _out():
        pltpu.sync_copy(acc_sh, out_hbm.at[c])


def _mm1_body(x_ref, w1_ref, c0_ref, c1_ref, g0_ref, g1_ref, dinv_ref):
    deg = c0_ref[0, 0, :] + c1_ref[0, 0, :] + 1.0
    dinv = lax.rsqrt(deg)
    h = jnp.dot(x_ref[...], w1_ref[...], preferred_element_type=jnp.float32)
    g = h * dinv[:, None]
    g0_ref[...] = g[:, :DH]
    g1_ref[...] = g[:, DH:]
    dinv_ref[0, 0, :] = dinv


def _l2_body(acc_ref, g0_ref, g1_ref, dinv_ref, b1_ref, w2_ref, q_ref):
    dinv = dinv_ref[0, 0, :]
    agg = jnp.concatenate(
        [acc_ref[0] + g0_ref[...], acc_ref[1] + g1_ref[...]], axis=1
    )
    pre = agg * dinv[:, None] + b1_ref[0][None, :]
    r = jnp.maximum(pre, 0.0)
    h2 = jnp.sum(r * w2_ref[0][None, :], axis=1)
    q_ref[0, 0, :] = h2 * dinv


def _final_body(s_ref, q_ref, dinv_ref, b2_ref, out_ref):
    tot = s_ref[0, 0, 0, :] + s_ref[1, 0, 0, :] + q_ref[0, 0, :]
    out_ref[0, 0, :] = jnp.tanh(dinv_ref[0, 0, :] * tot + b2_ref[0, 0])


_mm1_call = pl.pallas_call(
    _mm1_body,
    grid=(NB,),
    in_specs=[
        pl.BlockSpec((BLK, D), lambda i: (i, 0)),
        pl.BlockSpec((D, D), lambda i: (0, 0)),
        pl.BlockSpec((1, 1, BLK), lambda i: (i, 0, 0)),
        pl.BlockSpec((1, 1, BLK), lambda i: (i, 0, 0)),
    ],
    out_specs=[
        pl.BlockSpec((BLK, DH), lambda i: (i, 0)),
        pl.BlockSpec((BLK, DH), lambda i: (i, 0)),
        pl.BlockSpec((1, 1, BLK), lambda i: (i, 0, 0)),
    ],
    out_shape=[
        jax.ShapeDtypeStruct((N, DH), jnp.float32),
        jax.ShapeDtypeStruct((N, DH), jnp.float32),
        jax.ShapeDtypeStruct((NB, 1, BLK), jnp.float32),
    ],
)

_l2_call = pl.pallas_call(
    _l2_body,
    grid=(NB,),
    in_specs=[
        pl.BlockSpec((NC, BLK, DH), lambda i: (0, i, 0)),
        pl.BlockSpec((BLK, DH), lambda i: (i, 0)),
        pl.BlockSpec((BLK, DH), lambda i: (i, 0)),
        pl.BlockSpec((1, 1, BLK), lambda i: (i, 0, 0)),
        pl.BlockSpec((1, D), lambda i: (0, 0)),
        pl.BlockSpec((1, D), lambda i: (0, 0)),
    ],
    out_specs=pl.BlockSpec((1, 1, BLK), lambda i: (i, 0, 0)),
    out_shape=jax.ShapeDtypeStruct((NB, 1, BLK), jnp.float32),
)

_final_call = pl.pallas_call(
    _final_body,
    grid=(NB,),
    in_specs=[
        pl.BlockSpec((NC, 1, 1, BLK), lambda i: (0, i, 0, 0)),
        pl.BlockSpec((1, 1, BLK), lambda i: (i, 0, 0)),
        pl.BlockSpec((1, 1, BLK), lambda i: (i, 0, 0)),
        pl.BlockSpec((1, 1), lambda i: (0, 0)),
    ],
    out_specs=pl.BlockSpec((1, 1, BLK), lambda i: (i, 0, 0)),
    out_shape=jax.ShapeDtypeStruct((NB, 1, BLK), jnp.float32),
)


def kernel(x, edge_index, W1, b1, W2, b2):
    ei = edge_index.astype(jnp.int32)
    src2 = ei[0].reshape(ER, CH)
    dst2 = ei[1].reshape(ER, CH)

    counts = _count_k(dst2)
    c0 = counts[0].reshape(NB, 1, BLK)
    c1 = counts[1].reshape(NB, 1, BLK)
    g0, g1, dinv3 = _mm1_call(x, W1, c0, c1)
    acc = _agg_k(src2, dst2, g0, g1)
    q3 = _l2_call(acc, g0, g1, dinv3, b1.reshape(1, D), W2.reshape(1, D))
    s_part = _sagg_k(src2, dst2, q3.reshape(N))
    out3 = _final_call(
        s_part.reshape(NC, NB, 1, BLK), q3, dinv3, b2.reshape(1, 1)
    )
    return out3.reshape(N, 1)


# speedup vs baseline: 42.4475x; 1.6262x over previous
import functools

import jax
import jax.numpy as jnp
from jax import lax
from jax.experimental import pallas as pl
from jax.experimental.pallas import tpu as pltpu, tpu_sc as plsc

N = 10000
E = 320000
D = 128
NC, NS = 2, 16
NW = NC * NS
CH = 125
ER = E // CH
NCH = ER // NW
NPAD = 10240
SROW = NPAD // NS

BLK = 1000
NB = N // BLK

_mesh = plsc.VectorSubcoreMesh(
    core_axis_name="c", subcore_axis_name="s", num_cores=NC, num_subcores=NS
)


@functools.partial(
    pl.kernel,
    out_type=jax.ShapeDtypeStruct((NC, N), jnp.float32),
    mesh=_mesh,
    scratch_types=[
        pltpu.VMEM((NCH, CH), jnp.int32),
        pltpu.VMEM((128,), jnp.float32),
        pltpu.VMEM((N,), jnp.float32),
        pltpu.VMEM_SHARED((N,), jnp.float32),
    ],
)
def _count_k(dst_hbm, out_hbm, idx_v, ones_v, zero_v, cnt_sh):
    c = lax.axis_index("c")
    s = lax.axis_index("s")
    w = c * NS + s
    for k in range(128 // 16):
        ones_v[pl.ds(k * 16, 16)] = jnp.ones((16,), jnp.float32)

    @pl.when(s == 0)
    def _zero():
        def zb(i, carry):
            zero_v[pl.ds(i * 16, 16)] = jnp.zeros((16,), jnp.float32)
            return carry

        lax.fori_loop(0, N // 16, zb, 0)
        pltpu.sync_copy(zero_v, cnt_sh)

    plsc.subcore_barrier()
    pltpu.sync_copy(dst_hbm.at[pl.ds(w * NCH, NCH)], idx_v)

    def body(j, carry):
        pltpu.sync_copy(ones_v.at[pl.ds(0, CH)], cnt_sh.at[idx_v.at[j]], add=True)
        return carry

    lax.fori_loop(0, NCH, body, 0)
    plsc.subcore_barrier()

    @pl.when(s == 0)
    def _out():
        pltpu.sync_copy(cnt_sh, out_hbm.at[c])


DH = D // 2
ECH = ER // NS


@functools.partial(
    pl.kernel,
    out_type=jax.ShapeDtypeStruct((NC, NPAD, DH), jnp.float32),
    mesh=_mesh,
    scratch_types=[
        pltpu.VMEM((ECH, CH), jnp.int32),
        pltpu.VMEM((ECH, CH), jnp.int32),
        pltpu.VMEM((CH, DH), jnp.float32),
        pltpu.VMEM((CH, DH), jnp.float32),
        pltpu.VMEM((128, DH), jnp.float32),
        pltpu.VMEM_SHARED((NPAD, DH), jnp.float32),
        pltpu.SemaphoreType.DMA,
        pltpu.SemaphoreType.DMA,
    ],
    compiler_params=pltpu.CompilerParams(use_tc_tiling_on_sc=False),
)
def _agg_k(src_hbm, dst_hbm, g0_hbm, g1_hbm, out_hbm, src_v, dst_v, rows0_v, rows1_v, zbuf_v, acc_sh, sem0, sem1):
    c = lax.axis_index("c")
    s = lax.axis_index("s")

    def zr(i, carry):
        def zk(k, carry2):
            zbuf_v[i, pl.ds(k * 16, 16)] = jnp.zeros((16,), jnp.float32)
            return carry2

        lax.fori_loop(0, DH // 16, zk, 0)
        return carry

    lax.fori_loop(0, 128, zr, 0)
    for t in range(SROW // 128):
        pltpu.sync_copy(zbuf_v, acc_sh.at[pl.ds(s * SROW + t * 128, 128)])
    plsc.subcore_barrier()

    pltpu.sync_copy(src_hbm.at[pl.ds(s * ECH, ECH)], src_v)
    pltpu.sync_copy(dst_hbm.at[pl.ds(s * ECH, ECH)], dst_v)

    def _edge_loop(g_hbm):
        pltpu.async_copy(g_hbm.at[src_v.at[0]], rows0_v, sem0)
        pltpu.async_copy(g_hbm.at[src_v.at[1]], rows1_v, sem1)

        def body(jj, carry):
            j0 = 2 * jj
            pltpu.make_async_copy(g_hbm.at[src_v.at[j0]], rows0_v, sem0).wait()
            pltpu.sync_copy(rows0_v, acc_sh.at[dst_v.at[j0]], add=True)

            @pl.when(jj < ECH // 2 - 1)
            def _n0():
                pltpu.async_copy(g_hbm.at[src_v.at[j0 + 2]], rows0_v, sem0)

            pltpu.make_async_copy(g_hbm.at[src_v.at[j0 + 1]], rows1_v, sem1).wait()
            pltpu.sync_copy(rows1_v, acc_sh.at[dst_v.at[j0 + 1]], add=True)

            @pl.when(jj < ECH // 2 - 1)
            def _n1():
                pltpu.async_copy(g_hbm.at[src_v.at[j0 + 3]], rows1_v, sem1)

            return carry

        lax.fori_loop(0, ECH // 2, body, 0)

    @pl.when(c == 0)
    def _half0():
        _edge_loop(g0_hbm)

    @pl.when(c == 1)
    def _half1():
        _edge_loop(g1_hbm)

    plsc.subcore_barrier()
    pltpu.sync_copy(
        acc_sh.at[pl.ds(s * SROW, SROW)], out_hbm.at[c, pl.ds(s * SROW, SROW)]
    )


CH2 = 80
EPT = E // NW
NC2 = EPT // CH2


@functools.partial(
    pl.kernel,
    out_type=jax.ShapeDtypeStruct((NC, N), jnp.float32),
    mesh=_mesh,
    scratch_types=[
        pltpu.VMEM((EPT,), jnp.int32),
        pltpu.VMEM((NC2, CH2), jnp.int32),
        pltpu.VMEM((EPT,), jnp.float32),
        pltpu.VMEM((N,), jnp.float32),
        pltpu.VMEM((N,), jnp.float32),
        pltpu.VMEM_SHARED((N,), jnp.float32),
    ],
    compiler_params=pltpu.CompilerParams(needs_layout_passes=False),
)
def _sagg_k(src_hbm, dst_hbm, q_hbm, out_hbm, src_v, dst_v, vals_v, q_v, zero_v, acc_sh):
    c = lax.axis_index("c")
    s = lax.axis_index("s")
    w = c * NS + s

    @pl.when(s == 0)
    def _zero():
        def zb(i, carry):
            zero_v[pl.ds(i * 16, 16)] = jnp.zeros((16,), jnp.float32)
            return carry

        lax.fori_loop(0, N // 16, zb, 0)
        pltpu.sync_copy(zero_v, acc_sh)

    plsc.subcore_barrier()
    pltpu.sync_copy(q_hbm, q_v)
    pltpu.sync_copy(src_hbm.at[pl.ds(w * EPT, EPT)], src_v)
    pltpu.sync_copy(dst_hbm.at[w], dst_v)

    def gbody(i, carry):
        iv = src_v[pl.ds(i * 16, 16)]
        vals_v[pl.ds(i * 16, 16)] = plsc.load_gather(q_v, [iv])
        return carry

    lax.fori_loop(0, EPT // 16, gbody, 0)

    def sbody(j, carry):
        pltpu.sync_copy(vals_v.at[pl.ds(j * CH2, CH2)], acc_sh.at[dst_v.at[j]], add=True)
        return carry

    lax.fori_loop(0, NC2, sbody, 0)
    plsc.subcore_barrier()

    @pl.when(s == 0)
    def _out():
        pltpu.sync_copy(acc_sh, out_hbm.at[c])


def _mm1_body(x_ref, w1_ref, c0_ref, c1_ref, g0_ref, g1_ref, dinv_ref):
    deg = c0_ref[0, 0, :] + c1_ref[0, 0, :] + 1.0
    dinv = lax.rsqrt(deg)
    h = jnp.dot(x_ref[...], w1_ref[...], preferred_element_type=jnp.float32)
    g = h * dinv[:, None]
    g0_ref[...] = g[:, :DH]
    g1_ref[...] = g[:, DH:]
    dinv_ref[0, 0, :] = dinv


def _l2_body(acc_ref, g0_ref, g1_ref, dinv_ref, b1_ref, w2_ref, q_ref):
    dinv = dinv_ref[0, 0, :]
    agg = jnp.concatenate(
        [acc_ref[0] + g0_ref[...], acc_ref[1] + g1_ref[...]], axis=1
    )
    pre = agg * dinv[:, None] + b1_ref[0][None, :]
    r = jnp.maximum(pre, 0.0)
    h2 = jnp.sum(r * w2_ref[0][None, :], axis=1)
    q_ref[0, 0, :] = h2 * dinv


def _final_body(s_ref, q_ref, dinv_ref, b2_ref, out_ref):
    tot = s_ref[0, 0, 0, :] + s_ref[1, 0, 0, :] + q_ref[0, 0, :]
    out_ref[0, 0, :] = jnp.tanh(dinv_ref[0, 0, :] * tot + b2_ref[0, 0])


_mm1_call = pl.pallas_call(
    _mm1_body,
    grid=(NB,),
    in_specs=[
        pl.BlockSpec((BLK, D), lambda i: (i, 0)),
        pl.BlockSpec((D, D), lambda i: (0, 0)),
        pl.BlockSpec((1, 1, BLK), lambda i: (i, 0, 0)),
        pl.BlockSpec((1, 1, BLK), lambda i: (i, 0, 0)),
    ],
    out_specs=[
        pl.BlockSpec((BLK, DH), lambda i: (i, 0)),
        pl.BlockSpec((BLK, DH), lambda i: (i, 0)),
        pl.BlockSpec((1, 1, BLK), lambda i: (i, 0, 0)),
    ],
    out_shape=[
        jax.ShapeDtypeStruct((N, DH), jnp.float32),
        jax.ShapeDtypeStruct((N, DH), jnp.float32),
        jax.ShapeDtypeStruct((NB, 1, BLK), jnp.float32),
    ],
)

_l2_call = pl.pallas_call(
    _l2_body,
    grid=(NB,),
    in_specs=[
        pl.BlockSpec((NC, BLK, DH), lambda i: (0, i, 0)),
        pl.BlockSpec((BLK, DH), lambda i: (i, 0)),
        pl.BlockSpec((BLK, DH), lambda i: (i, 0)),
        pl.BlockSpec((1, 1, BLK), lambda i: (i, 0, 0)),
        pl.BlockSpec((1, D), lambda i: (0, 0)),
        pl.BlockSpec((1, D), lambda i: (0, 0)),
    ],
    out_specs=pl.BlockSpec((1, 1, BLK), lambda i: (i, 0, 0)),
    out_shape=jax.ShapeDtypeStruct((NB, 1, BLK), jnp.float32),
)

_final_call = pl.pallas_call(
    _final_body,
    grid=(NB,),
    in_specs=[
        pl.BlockSpec((NC, 1, 1, BLK), lambda i: (0, i, 0, 0)),
        pl.BlockSpec((1, 1, BLK), lambda i: (i, 0, 0)),
        pl.BlockSpec((1, 1, BLK), lambda i: (i, 0, 0)),
        pl.BlockSpec((1, 1), lambda i: (0, 0)),
    ],
    out_specs=pl.BlockSpec((1, 1, BLK), lambda i: (i, 0, 0)),
    out_shape=jax.ShapeDtypeStruct((NB, 1, BLK), jnp.float32),
)


def kernel(x, edge_index, W1, b1, W2, b2):
    ei = edge_index.astype(jnp.int32)
    src2 = ei[0].reshape(ER, CH)
    dst2 = ei[1].reshape(ER, CH)

    counts = _count_k(dst2)
    c0 = counts[0].reshape(NB, 1, BLK)
    c1 = counts[1].reshape(NB, 1, BLK)
    g0, g1, dinv3 = _mm1_call(x, W1, c0, c1)
    acc = _agg_k(src2, dst2, g0, g1)
    q3 = _l2_call(acc, g0, g1, dinv3, b1.reshape(1, D), W2.reshape(1, D))
    s_part = _sagg_k(ei[0], ei[1].reshape(NW, NC2, CH2), q3.reshape(N))
    out3 = _final_call(
        s_part.reshape(NC, NB, 1, BLK), q3, dinv3, b2.reshape(1, 1)
    )
    return out3.reshape(N, 1)
